# baseline (speedup 1.0000x reference)
import functools

import jax
import jax.numpy as jnp
from jax import lax
from jax.experimental import pallas as pl
from jax.experimental.pallas import tpu as pltpu
from jax.experimental.pallas import tpu_sc as plsc

SIGMA = 2.0
EPS = 1e-5
NEG_SLOPE = 0.01
GROUPS = 32


def _make_sc_gather(n_rows, n_edges, c_feat, feat_dtype):
    info = plsc.get_sparse_core_info()
    nc, ns = info.num_cores, info.num_subcores
    nw = nc * ns
    assert n_edges % nw == 0
    per_w = n_edges // nw
    ch = 128
    n_full = per_w // ch
    rem = per_w - n_full * ch
    assert rem % 16 == 0 and n_full % 3 == 0 and n_full >= 6
    n_grp = n_full // 3
    mesh = plsc.VectorSubcoreMesh(core_axis_name="c", subcore_axis_name="s")

    nb = 3
    scratch = (
        [pltpu.VMEM((ch,), jnp.int32)] * nb
        + [pltpu.VMEM((ch, c_feat), feat_dtype)] * nb
        + [pltpu.VMEM((ch,), jnp.float32)] * (3 * nb)
        + [
            pltpu.VMEM((n_rows,), jnp.float32),
            pltpu.VMEM((n_rows,), jnp.float32),
            pltpu.VMEM((n_rows,), jnp.float32),
        ]
        + [pltpu.SemaphoreType.DMA] * (3 * nb)
    )

    @functools.partial(
        pl.kernel,
        mesh=mesh,
        compiler_params=pltpu.CompilerParams(needs_layout_passes=False),
        out_type=[
            jax.ShapeDtypeStruct((n_edges, c_feat), feat_dtype),
            jax.ShapeDtypeStruct((n_edges,), jnp.float32),
            jax.ShapeDtypeStruct((n_edges,), jnp.float32),
            jax.ShapeDtypeStruct((n_edges,), jnp.float32),
        ],
        scratch_types=scratch,
    )
    def gather(feats_hbm, xs_hbm, ys_hbm, zs_hbm, idx_hbm,
               gf_hbm, gx_hbm, gy_hbm, gz_hbm, *bufs):
        idx_v = bufs[0:nb]
        rows_v = bufs[nb:2 * nb]
        xb_v = bufs[2 * nb:3 * nb]
        yb_v = bufs[3 * nb:4 * nb]
        zb_v = bufs[4 * nb:5 * nb]
        xt_v, yt_v, zt_v = bufs[5 * nb:5 * nb + 3]
        sem_i = bufs[5 * nb + 3:5 * nb + 3 + nb]
        sem_g = bufs[5 * nb + 3 + nb:5 * nb + 3 + 2 * nb]
        sem_o = bufs[5 * nb + 3 + 2 * nb:5 * nb + 3 + 3 * nb]

        wid = lax.axis_index("s") * nc + lax.axis_index("c")
        base0 = wid * per_w
        pltpu.sync_copy(xs_hbm, xt_v)
        pltpu.sync_copy(ys_hbm, yt_v)
        pltpu.sync_copy(zs_hbm, zt_v)

        def idx_cp(c, b):
            return pltpu.make_async_copy(
                idx_hbm.at[pl.ds(base0 + c * ch, ch)], idx_v[b], sem_i[b])

        def gather_cp(b):
            return pltpu.make_async_copy(
                feats_hbm.at[idx_v[b]], rows_v[b], sem_g[b])

        def out_cps(c, b):
            sl = pl.ds(base0 + c * ch, ch)
            return (
                pltpu.make_async_copy(rows_v[b], gf_hbm.at[sl], sem_o[b]),
                pltpu.make_async_copy(xb_v[b], gx_hbm.at[sl], sem_o[b]),
                pltpu.make_async_copy(yb_v[b], gy_hbm.at[sl], sem_o[b]),
                pltpu.make_async_copy(zb_v[b], gz_hbm.at[sl], sem_o[b]),
            )

        def xyz_comp(b):
            def xyz_body(j, carry):
                row16 = idx_v[b][pl.ds(j * 16, 16)]
                xb_v[b][pl.ds(j * 16, 16)] = plsc.load_gather(xt_v, [row16])
                yb_v[b][pl.ds(j * 16, 16)] = plsc.load_gather(yt_v, [row16])
                zb_v[b][pl.ds(j * 16, 16)] = plsc.load_gather(zt_v, [row16])
                return carry
            lax.fori_loop(0, ch // 16, xyz_body, 0)

        idx_cp(0, 0).start()
        idx_cp(0, 0).wait()
        gather_cp(0).start()
        xyz_comp(0)
        idx_cp(1, 1).start()
        idx_cp(1, 1).wait()
        gather_cp(1).start()
        xyz_comp(1)
        idx_cp(2, 2).start()
        gather_cp(0).wait()
        for cp in out_cps(0, 0):
            cp.start()
        idx_cp(2, 2).wait()
        gather_cp(2).start()
        xyz_comp(2)
        idx_cp(3, 0).start()
        gather_cp(1).wait()
        for cp in out_cps(1, 1):
            cp.start()

        def steady(j, carry):
            for b in range(3):
                c = 3 * j + b
                for cp in out_cps(c - 3, b):
                    cp.wait()
                idx_cp(c, b).wait()
                gather_cp(b).start()
                xyz_comp(b)
                bn1 = (b + 1) % 3
                if b < 2:
                    idx_cp(c + 1, bn1).start()
                else:
                    @pl.when(j < n_grp - 1)
                    def _():
                        idx_cp(c + 1, bn1).start()
                bp = (b + 2) % 3
                gather_cp(bp).wait()
                for cp in out_cps(c - 1, bp):
                    cp.start()
            return carry

        lax.fori_loop(1, n_grp, steady, 0)

        last = n_full - 1
        gather_cp(2).wait()
        for cp in out_cps(last, 2):
            cp.start()
        for b, c in ((0, last - 2), (1, last - 1), (2, last)):
            for cp in out_cps(c, b):
                cp.wait()

        if rem:
            base = base0 + n_full * ch
            sl = pl.ds(base, rem)
            pltpu.sync_copy(idx_hbm.at[sl], idx_v[0].at[pl.ds(0, rem)])
            pltpu.async_copy(
                feats_hbm.at[idx_v[0].at[pl.ds(0, rem)]],
                rows_v[0].at[pl.ds(0, rem)], sem_g[0]).wait()

            def rem_body(j, carry):
                row16 = idx_v[0][pl.ds(j * 16, 16)]
                xb_v[0][pl.ds(j * 16, 16)] = plsc.load_gather(xt_v, [row16])
                yb_v[0][pl.ds(j * 16, 16)] = plsc.load_gather(yt_v, [row16])
                zb_v[0][pl.ds(j * 16, 16)] = plsc.load_gather(zt_v, [row16])
                return carry
            lax.fori_loop(0, rem // 16, rem_body, 0)
            pltpu.sync_copy(rows_v[0].at[pl.ds(0, rem)], gf_hbm.at[sl])
            pltpu.sync_copy(xb_v[0].at[pl.ds(0, rem)], gx_hbm.at[sl])
            pltpu.sync_copy(yb_v[0].at[pl.ds(0, rem)], gy_hbm.at[sl])
            pltpu.sync_copy(zb_v[0].at[pl.ds(0, rem)], gz_hbm.at[sl])

    return gather


def _tc_main_body(gf_ref, gx_ref, gy_ref, gz_ref, qx_ref, qy_ref, qz_ref,
                  kp_ref, w_ref, out_ref, sums_ref, gall_ref,
                  *, bq, k, c_in, c_out, p):
    i = pl.program_id(0)
    ng = (bq * k) // 256

    kxb = jnp.broadcast_to(kp_ref[:, 0:1], (16, 256))
    kyb = jnp.broadcast_to(kp_ref[:, 1:2], (16, 256))
    kzb = jnp.broadcast_to(kp_ref[:, 2:3], (16, 256))
    sub = lax.broadcasted_iota(jnp.int32, (128, 256), 0)
    lan = lax.broadcasted_iota(jnp.int32, (128, 256), 1)
    mask = ((sub % 8) == (lan // 32)).astype(jnp.float32)

    for g in range(ng):
        ex = gx_ref[0, g:g + 1, :] - qx_ref[0, g:g + 1, :]
        ey = gy_ref[0, g:g + 1, :] - qy_ref[0, g:g + 1, :]
        ez = gz_ref[0, g:g + 1, :] - qz_ref[0, g:g + 1, :]
        tx = jnp.broadcast_to(ex, (16, 256)) - kxb
        ty = jnp.broadcast_to(ey, (16, 256)) - kyb
        tz = jnp.broadcast_to(ez, (16, 256)) - kzb
        sq = tx * tx + ty * ty + tz * tz
        dist = sq * lax.rsqrt(sq + 1e-12)
        infl = jnp.maximum(1.0 - dist * (1.0 / SIGMA), 0.0)
        rep = jnp.broadcast_to(
            infl.reshape(16, 1, 256), (16, 8, 256)).reshape(128, 256)
        bdt = rep * mask
        g2 = jnp.dot(bdt, gf_ref[g * 256:(g + 1) * 256, :],
                     preferred_element_type=jnp.float32)
        for pp in range(p):
            gall_ref[pp * bq + g * 8:pp * bq + g * 8 + 8, :] = (
                g2[pp * 8:(pp + 1) * 8, :])

    acc = jnp.zeros((bq, c_out), jnp.float32)
    for pp in range(p):
        acc = acc + jnp.dot(gall_ref[pp * bq:(pp + 1) * bq, :], w_ref[pp],
                            preferred_element_type=jnp.float32)
    out_ref[...] = acc

    @pl.when(i == 0)
    def _():
        sums_ref[...] = jnp.zeros_like(sums_ref)

    sums_ref[0:1, :] += jnp.sum(acc, axis=0, keepdims=True)
    sums_ref[1:2, :] += jnp.sum(acc * acc, axis=0, keepdims=True)


def _tc_norm_body(x_ref, *refs, n, c_out):
    sums_refs = refs[:-3]
    gamma_ref, beta_ref, y_ref = refs[-3:]
    cpg = c_out // GROUPS
    r = lax.broadcasted_iota(jnp.int32, (c_out, c_out), 0) // cpg
    c = lax.broadcasted_iota(jnp.int32, (c_out, c_out), 1) // cpg
    gmat = (r == c).astype(jnp.float32)
    csum = sum(s[0:1, :] for s in sums_refs)
    csq = sum(s[1:2, :] for s in sums_refs)
    denom = 1.0 / (n * cpg)
    mean = jnp.dot(csum, gmat, preferred_element_type=jnp.float32) * denom
    e2 = jnp.dot(csq, gmat, preferred_element_type=jnp.float32) * denom
    var = e2 - mean * mean
    a = gamma_ref[...] * lax.rsqrt(var + EPS)
    b = beta_ref[...] - mean * a
    y = x_ref[...] * a + b
    y_ref[...] = jnp.where(y >= 0, y, NEG_SLOPE * y)


def kernel(q_points, s_points, s_feats, neighbor_indices, kernel_points,
           weights, gamma, beta):
    n, c_in = s_feats.shape
    nq, k = neighbor_indices.shape
    p, _, c_out = weights.shape
    n_edges = nq * k

    idx_flat = neighbor_indices.reshape(n_edges)
    kp_t = jnp.full((16, 8), 1e4, jnp.float32).at[0:p, 0:3].set(kernel_points)
    bq = 200
    n_slices = 5
    nq_s = nq // n_slices
    ne_s = n_edges // n_slices
    n_blocks = nq_s // bq
    ng = (bq * k) // 256
    e3 = (n_blocks, ng, 256)

    gather = _make_sc_gather(n, ne_s, c_in, jnp.float32)
    sx, sy, sz = s_points[:, 0], s_points[:, 1], s_points[:, 2]

    outs, sums_l = [], []
    for s in range(n_slices):
        gf, gxf, gyf, gzf = gather(
            s_feats, sx, sy, sz, idx_flat[s * ne_s:(s + 1) * ne_s])
        gx = gxf.reshape(e3)
        gy = gyf.reshape(e3)
        gz = gzf.reshape(e3)
        qsl = q_points[s * nq_s:(s + 1) * nq_s]
        qx = jnp.repeat(qsl[:, 0], k).reshape(e3)
        qy = jnp.repeat(qsl[:, 1], k).reshape(e3)
        qz = jnp.repeat(qsl[:, 2], k).reshape(e3)

        out_s, sums_s = pl.pallas_call(
            functools.partial(_tc_main_body, bq=bq, k=k, c_in=c_in,
                              c_out=c_out, p=p),
            grid=(n_blocks,),
            in_specs=[
                pl.BlockSpec((bq * k, c_in), lambda i: (i, 0)),
                pl.BlockSpec((1, ng, 256), lambda i: (i, 0, 0)),
                pl.BlockSpec((1, ng, 256), lambda i: (i, 0, 0)),
                pl.BlockSpec((1, ng, 256), lambda i: (i, 0, 0)),
                pl.BlockSpec((1, ng, 256), lambda i: (i, 0, 0)),
                pl.BlockSpec((1, ng, 256), lambda i: (i, 0, 0)),
                pl.BlockSpec((1, ng, 256), lambda i: (i, 0, 0)),
                pl.BlockSpec((16, 8), lambda i: (0, 0)),
                pl.BlockSpec((p, c_in, c_out), lambda i: (0, 0, 0)),
            ],
            out_specs=[
                pl.BlockSpec((bq, c_out), lambda i: (i, 0)),
                pl.BlockSpec((8, c_out), lambda i: (0, 0)),
            ],
            out_shape=[
                jax.ShapeDtypeStruct((nq_s, c_out), jnp.float32),
                jax.ShapeDtypeStruct((8, c_out), jnp.float32),
            ],
            scratch_shapes=[pltpu.VMEM((p * bq, c_in), jnp.float32)],
        )(gf, gx, gy, gz, qx, qy, qz, kp_t, weights)
        outs.append(out_s)
        sums_l.append(sums_s)

    out = jnp.concatenate(outs, axis=0)

    bn = 1000
    y = pl.pallas_call(
        functools.partial(_tc_norm_body, n=nq, c_out=c_out),
        grid=(nq // bn,),
        in_specs=[
            pl.BlockSpec((bn, c_out), lambda i: (i, 0)),
        ] + [
            pl.BlockSpec((8, c_out), lambda i: (0, 0))
            for _ in range(n_slices)
        ] + [
            pl.BlockSpec((1, c_out), lambda i: (0, 0)),
            pl.BlockSpec((1, c_out), lambda i: (0, 0)),
        ],
        out_specs=pl.BlockSpec((bn, c_out), lambda i: (i, 0)),
        out_shape=jax.ShapeDtypeStruct((nq, c_out), jnp.float32),
    )(out, *sums_l, gamma.reshape(1, c_out), beta.reshape(1, c_out))

    return y


# speedup vs baseline: 1.1690x; 1.1690x over previous
import functools

import jax
import jax.numpy as jnp
from jax import lax
from jax.experimental import pallas as pl
from jax.experimental.pallas import tpu as pltpu
from jax.experimental.pallas import tpu_sc as plsc

SIGMA = 2.0
EPS = 1e-5
NEG_SLOPE = 0.01
GROUPS = 32


def _make_sc_gather(n_rows, n_edges, c_feat, feat_dtype):
    info = plsc.get_sparse_core_info()
    nc, ns = info.num_cores, info.num_subcores
    nw = nc * ns
    assert n_edges % nw == 0
    per_w = n_edges // nw
    ch = 128
    n_full = per_w // ch
    rem = per_w - n_full * ch
    assert rem % 16 == 0 and n_full % 3 == 0 and n_full >= 6
    n_grp = n_full // 3
    mesh = plsc.VectorSubcoreMesh(core_axis_name="c", subcore_axis_name="s")

    nb = 3
    scratch = (
        [pltpu.VMEM((ch,), jnp.int32)] * nb
        + [pltpu.VMEM((ch, c_feat), feat_dtype)] * nb
        + [pltpu.VMEM((ch,), jnp.float32)] * (3 * nb)
        + [
            pltpu.VMEM((n_rows,), jnp.float32),
            pltpu.VMEM((n_rows,), jnp.float32),
            pltpu.VMEM((n_rows,), jnp.float32),
        ]
        + [pltpu.SemaphoreType.DMA] * (3 * nb)
    )

    @functools.partial(
        pl.kernel,
        mesh=mesh,
        compiler_params=pltpu.CompilerParams(needs_layout_passes=False),
        out_type=[
            jax.ShapeDtypeStruct((n_edges, c_feat), feat_dtype),
            jax.ShapeDtypeStruct((n_edges,), jnp.float32),
            jax.ShapeDtypeStruct((n_edges,), jnp.float32),
            jax.ShapeDtypeStruct((n_edges,), jnp.float32),
        ],
        scratch_types=scratch,
    )
    def gather(feats_hbm, xs_hbm, ys_hbm, zs_hbm, idx_hbm,
               gf_hbm, gx_hbm, gy_hbm, gz_hbm, *bufs):
        idx_v = bufs[0:nb]
        rows_v = bufs[nb:2 * nb]
        xb_v = bufs[2 * nb:3 * nb]
        yb_v = bufs[3 * nb:4 * nb]
        zb_v = bufs[4 * nb:5 * nb]
        xt_v, yt_v, zt_v = bufs[5 * nb:5 * nb + 3]
        sem_i = bufs[5 * nb + 3:5 * nb + 3 + nb]
        sem_g = bufs[5 * nb + 3 + nb:5 * nb + 3 + 2 * nb]
        sem_o = bufs[5 * nb + 3 + 2 * nb:5 * nb + 3 + 3 * nb]

        wid = lax.axis_index("s") * nc + lax.axis_index("c")
        base0 = wid * per_w
        pltpu.sync_copy(xs_hbm, xt_v)
        pltpu.sync_copy(ys_hbm, yt_v)
        pltpu.sync_copy(zs_hbm, zt_v)

        def idx_cp(c, b):
            return pltpu.make_async_copy(
                idx_hbm.at[pl.ds(base0 + c * ch, ch)], idx_v[b], sem_i[b])

        def gather_cp(b):
            return pltpu.make_async_copy(
                feats_hbm.at[idx_v[b]], rows_v[b], sem_g[b])

        def out_cps(c, b):
            sl = pl.ds(base0 + c * ch, ch)
            return (
                pltpu.make_async_copy(rows_v[b], gf_hbm.at[sl], sem_o[b]),
                pltpu.make_async_copy(xb_v[b], gx_hbm.at[sl], sem_o[b]),
                pltpu.make_async_copy(yb_v[b], gy_hbm.at[sl], sem_o[b]),
                pltpu.make_async_copy(zb_v[b], gz_hbm.at[sl], sem_o[b]),
            )

        def xyz_comp(b):
            def xyz_body(j, carry):
                row16 = idx_v[b][pl.ds(j * 16, 16)]
                xb_v[b][pl.ds(j * 16, 16)] = plsc.load_gather(xt_v, [row16])
                yb_v[b][pl.ds(j * 16, 16)] = plsc.load_gather(yt_v, [row16])
                zb_v[b][pl.ds(j * 16, 16)] = plsc.load_gather(zt_v, [row16])
                return carry
            lax.fori_loop(0, ch // 16, xyz_body, 0)

        idx_cp(0, 0).start()
        idx_cp(0, 0).wait()
        gather_cp(0).start()
        xyz_comp(0)
        idx_cp(1, 1).start()
        idx_cp(1, 1).wait()
        gather_cp(1).start()
        xyz_comp(1)
        idx_cp(2, 2).start()
        gather_cp(0).wait()
        for cp in out_cps(0, 0):
            cp.start()
        idx_cp(2, 2).wait()
        gather_cp(2).start()
        xyz_comp(2)
        idx_cp(3, 0).start()
        gather_cp(1).wait()
        for cp in out_cps(1, 1):
            cp.start()

        def steady(j, carry):
            for b in range(3):
                c = 3 * j + b
                for cp in out_cps(c - 3, b):
                    cp.wait()
                idx_cp(c, b).wait()
                gather_cp(b).start()
                xyz_comp(b)
                bn1 = (b + 1) % 3
                if b < 2:
                    idx_cp(c + 1, bn1).start()
                else:
                    @pl.when(j < n_grp - 1)
                    def _():
                        idx_cp(c + 1, bn1).start()
                bp = (b + 2) % 3
                gather_cp(bp).wait()
                for cp in out_cps(c - 1, bp):
                    cp.start()
            return carry

        lax.fori_loop(1, n_grp, steady, 0)

        last = n_full - 1
        gather_cp(2).wait()
        for cp in out_cps(last, 2):
            cp.start()
        for b, c in ((0, last - 2), (1, last - 1), (2, last)):
            for cp in out_cps(c, b):
                cp.wait()

        if rem:
            base = base0 + n_full * ch
            sl = pl.ds(base, rem)
            pltpu.sync_copy(idx_hbm.at[sl], idx_v[0].at[pl.ds(0, rem)])
            pltpu.async_copy(
                feats_hbm.at[idx_v[0].at[pl.ds(0, rem)]],
                rows_v[0].at[pl.ds(0, rem)], sem_g[0]).wait()

            def rem_body(j, carry):
                row16 = idx_v[0][pl.ds(j * 16, 16)]
                xb_v[0][pl.ds(j * 16, 16)] = plsc.load_gather(xt_v, [row16])
                yb_v[0][pl.ds(j * 16, 16)] = plsc.load_gather(yt_v, [row16])
                zb_v[0][pl.ds(j * 16, 16)] = plsc.load_gather(zt_v, [row16])
                return carry
            lax.fori_loop(0, rem // 16, rem_body, 0)
            pltpu.sync_copy(rows_v[0].at[pl.ds(0, rem)], gf_hbm.at[sl])
            pltpu.sync_copy(xb_v[0].at[pl.ds(0, rem)], gx_hbm.at[sl])
            pltpu.sync_copy(yb_v[0].at[pl.ds(0, rem)], gy_hbm.at[sl])
            pltpu.sync_copy(zb_v[0].at[pl.ds(0, rem)], gz_hbm.at[sl])

    return gather


def _tc_main_body(gf_ref, gx_ref, gy_ref, gz_ref, qx_ref, qy_ref, qz_ref,
                  kp_ref, w_ref, out_ref, sums_ref, gall_ref,
                  *, bq, k, c_in, c_out, p):
    i = pl.program_id(0)
    ng = (bq * k) // 256

    kxb = jnp.broadcast_to(kp_ref[:, 0:1], (16, 256))
    kyb = jnp.broadcast_to(kp_ref[:, 1:2], (16, 256))
    kzb = jnp.broadcast_to(kp_ref[:, 2:3], (16, 256))
    sub = lax.broadcasted_iota(jnp.int32, (128, 256), 0)
    lan = lax.broadcasted_iota(jnp.int32, (128, 256), 1)
    mask = ((sub % 8) == (lan // 32)).astype(jnp.float32)

    for g in range(ng):
        ex = gx_ref[0, g:g + 1, :] - qx_ref[0, g:g + 1, :]
        ey = gy_ref[0, g:g + 1, :] - qy_ref[0, g:g + 1, :]
        ez = gz_ref[0, g:g + 1, :] - qz_ref[0, g:g + 1, :]
        tx = jnp.broadcast_to(ex, (16, 256)) - kxb
        ty = jnp.broadcast_to(ey, (16, 256)) - kyb
        tz = jnp.broadcast_to(ez, (16, 256)) - kzb
        sq = tx * tx + ty * ty + tz * tz
        dist = sq * lax.rsqrt(sq + 1e-12)
        infl = jnp.maximum(1.0 - dist * (1.0 / SIGMA), 0.0)
        rep = jnp.broadcast_to(
            infl.reshape(16, 1, 256), (16, 8, 256)).reshape(128, 256)
        bdt = rep * mask
        g2 = jnp.dot(bdt, gf_ref[g * 256:(g + 1) * 256, :],
                     preferred_element_type=jnp.float32)
        for pp in range(p):
            gall_ref[pp * bq + g * 8:pp * bq + g * 8 + 8, :] = (
                g2[pp * 8:(pp + 1) * 8, :])

    acc = jnp.zeros((bq, c_out), jnp.float32)
    for pp in range(p):
        acc = acc + jnp.dot(gall_ref[pp * bq:(pp + 1) * bq, :], w_ref[pp],
                            preferred_element_type=jnp.float32)
    out_ref[...] = acc

    @pl.when(i == 0)
    def _():
        sums_ref[...] = jnp.zeros_like(sums_ref)

    sums_ref[0:1, :] += jnp.sum(acc, axis=0, keepdims=True)
    sums_ref[1:2, :] += jnp.sum(acc * acc, axis=0, keepdims=True)


def _tc_norm_body(x_ref, *refs, n, c_out):
    sums_refs = refs[:-3]
    gamma_ref, beta_ref, y_ref = refs[-3:]
    cpg = c_out // GROUPS
    r = lax.broadcasted_iota(jnp.int32, (c_out, c_out), 0) // cpg
    c = lax.broadcasted_iota(jnp.int32, (c_out, c_out), 1) // cpg
    gmat = (r == c).astype(jnp.float32)
    csum = sum(s[0:1, :] for s in sums_refs)
    csq = sum(s[1:2, :] for s in sums_refs)
    denom = 1.0 / (n * cpg)
    mean = jnp.dot(csum, gmat, preferred_element_type=jnp.float32) * denom
    e2 = jnp.dot(csq, gmat, preferred_element_type=jnp.float32) * denom
    var = e2 - mean * mean
    a = gamma_ref[...] * lax.rsqrt(var + EPS)
    b = beta_ref[...] - mean * a
    y = x_ref[...] * a + b
    y_ref[...] = jnp.where(y >= 0, y, NEG_SLOPE * y)


def kernel(q_points, s_points, s_feats, neighbor_indices, kernel_points,
           weights, gamma, beta):
    n, c_in = s_feats.shape
    nq, k = neighbor_indices.shape
    p, _, c_out = weights.shape
    n_edges = nq * k

    idx_flat = neighbor_indices.reshape(n_edges)
    kp_t = jnp.full((16, 8), 1e4, jnp.float32).at[0:p, 0:3].set(kernel_points)
    bq = 200
    n_slices = 5
    nq_s = nq // n_slices
    ne_s = n_edges // n_slices
    n_blocks = nq_s // bq
    ng = (bq * k) // 256
    e3 = (n_blocks, ng, 256)

    gather = _make_sc_gather(n, ne_s, c_in, jnp.float32)
    sx, sy, sz = s_points[:, 0], s_points[:, 1], s_points[:, 2]

    outs, sums_l = [], []
    for s in range(n_slices):
        idx_s = idx_flat[s * ne_s:(s + 1) * ne_s]
        if s >= 2:
            idx_s, _ = lax.optimization_barrier((idx_s, outs[s - 2]))
        gf, gxf, gyf, gzf = gather(s_feats, sx, sy, sz, idx_s)
        gx = gxf.reshape(e3)
        gy = gyf.reshape(e3)
        gz = gzf.reshape(e3)
        qsl = q_points[s * nq_s:(s + 1) * nq_s]
        qx = jnp.repeat(qsl[:, 0], k).reshape(e3)
        qy = jnp.repeat(qsl[:, 1], k).reshape(e3)
        qz = jnp.repeat(qsl[:, 2], k).reshape(e3)

        out_s, sums_s = pl.pallas_call(
            functools.partial(_tc_main_body, bq=bq, k=k, c_in=c_in,
                              c_out=c_out, p=p),
            grid=(n_blocks,),
            in_specs=[
                pl.BlockSpec((bq * k, c_in), lambda i: (i, 0)),
                pl.BlockSpec((1, ng, 256), lambda i: (i, 0, 0)),
                pl.BlockSpec((1, ng, 256), lambda i: (i, 0, 0)),
                pl.BlockSpec((1, ng, 256), lambda i: (i, 0, 0)),
                pl.BlockSpec((1, ng, 256), lambda i: (i, 0, 0)),
                pl.BlockSpec((1, ng, 256), lambda i: (i, 0, 0)),
                pl.BlockSpec((1, ng, 256), lambda i: (i, 0, 0)),
                pl.BlockSpec((16, 8), lambda i: (0, 0)),
                pl.BlockSpec((p, c_in, c_out), lambda i: (0, 0, 0)),
            ],
            out_specs=[
                pl.BlockSpec((bq, c_out), lambda i: (i, 0)),
                pl.BlockSpec((8, c_out), lambda i: (0, 0)),
            ],
            out_shape=[
                jax.ShapeDtypeStruct((nq_s, c_out), jnp.float32),
                jax.ShapeDtypeStruct((8, c_out), jnp.float32),
            ],
            scratch_shapes=[pltpu.VMEM((p * bq, c_in), jnp.float32)],
        )(gf, gx, gy, gz, qx, qy, qz, kp_t, weights)
        outs.append(out_s)
        sums_l.append(sums_s)

    out = jnp.concatenate(outs, axis=0)

    bn = 1000
    y = pl.pallas_call(
        functools.partial(_tc_norm_body, n=nq, c_out=c_out),
        grid=(nq // bn,),
        in_specs=[
            pl.BlockSpec((bn, c_out), lambda i: (i, 0)),
        ] + [
            pl.BlockSpec((8, c_out), lambda i: (0, 0))
            for _ in range(n_slices)
        ] + [
            pl.BlockSpec((1, c_out), lambda i: (0, 0)),
            pl.BlockSpec((1, c_out), lambda i: (0, 0)),
        ],
        out_specs=pl.BlockSpec((bn, c_out), lambda i: (i, 0)),
        out_shape=jax.ShapeDtypeStruct((nq, c_out), jnp.float32),
    )(out, *sums_l, gamma.reshape(1, c_out), beta.reshape(1, c_out))

    return y


# speedup vs baseline: 1.2928x; 1.1059x over previous
import functools

import jax
import jax.numpy as jnp
from jax import lax
from jax.experimental import pallas as pl
from jax.experimental.pallas import tpu as pltpu
from jax.experimental.pallas import tpu_sc as plsc

SIGMA = 2.0
EPS = 1e-5
NEG_SLOPE = 0.01
GROUPS = 32


def _make_sc_gather(n_rows, n_edges, c_feat, feat_dtype):
    info = plsc.get_sparse_core_info()
    nc, ns = info.num_cores, info.num_subcores
    nw = nc * ns
    assert n_edges % nw == 0
    per_w = n_edges // nw
    ch = 128
    n_full = per_w // ch
    rem = per_w - n_full * ch
    assert rem % 8 == 0 and n_full % 3 == 0 and n_full >= 6
    n_grp = n_full // 3
    mesh = plsc.VectorSubcoreMesh(core_axis_name="c", subcore_axis_name="s")

    nb = 3
    scratch = (
        [pltpu.VMEM((ch,), jnp.int32)] * nb
        + [pltpu.VMEM((ch, c_feat), feat_dtype)] * nb
        + [pltpu.VMEM((ch,), jnp.float32)] * (3 * nb)
        + [
            pltpu.VMEM((n_rows,), jnp.float32),
            pltpu.VMEM((n_rows,), jnp.float32),
            pltpu.VMEM((n_rows,), jnp.float32),
        ]
        + [pltpu.SemaphoreType.DMA] * (3 * nb)
    )

    @functools.partial(
        pl.kernel,
        mesh=mesh,
        compiler_params=pltpu.CompilerParams(needs_layout_passes=False),
        out_type=[
            jax.ShapeDtypeStruct((n_edges, c_feat), feat_dtype),
            jax.ShapeDtypeStruct((n_edges,), jnp.float32),
            jax.ShapeDtypeStruct((n_edges,), jnp.float32),
            jax.ShapeDtypeStruct((n_edges,), jnp.float32),
        ],
        scratch_types=scratch,
    )
    def gather(feats_hbm, xs_hbm, ys_hbm, zs_hbm, idx_hbm,
               gf_hbm, gx_hbm, gy_hbm, gz_hbm, *bufs):
        idx_v = bufs[0:nb]
        rows_v = bufs[nb:2 * nb]
        xb_v = bufs[2 * nb:3 * nb]
        yb_v = bufs[3 * nb:4 * nb]
        zb_v = bufs[4 * nb:5 * nb]
        xt_v, yt_v, zt_v = bufs[5 * nb:5 * nb + 3]
        sem_i = bufs[5 * nb + 3:5 * nb + 3 + nb]
        sem_g = bufs[5 * nb + 3 + nb:5 * nb + 3 + 2 * nb]
        sem_o = bufs[5 * nb + 3 + 2 * nb:5 * nb + 3 + 3 * nb]

        wid = lax.axis_index("s") * nc + lax.axis_index("c")
        base0 = wid * per_w
        pltpu.sync_copy(xs_hbm, xt_v)
        pltpu.sync_copy(ys_hbm, yt_v)
        pltpu.sync_copy(zs_hbm, zt_v)

        def idx_cp(c, b):
            return pltpu.make_async_copy(
                idx_hbm.at[pl.ds(base0 + c * ch, ch)], idx_v[b], sem_i[b])

        def gather_cp(b):
            return pltpu.make_async_copy(
                feats_hbm.at[idx_v[b]], rows_v[b], sem_g[b])

        def out_cps(c, b):
            sl = pl.ds(base0 + c * ch, ch)
            return (
                pltpu.make_async_copy(rows_v[b], gf_hbm.at[sl], sem_o[b]),
                pltpu.make_async_copy(xb_v[b], gx_hbm.at[sl], sem_o[b]),
                pltpu.make_async_copy(yb_v[b], gy_hbm.at[sl], sem_o[b]),
                pltpu.make_async_copy(zb_v[b], gz_hbm.at[sl], sem_o[b]),
            )

        def xyz_comp(b):
            def xyz_body(j, carry):
                row16 = idx_v[b][pl.ds(j * 16, 16)]
                xb_v[b][pl.ds(j * 16, 16)] = plsc.load_gather(xt_v, [row16])
                yb_v[b][pl.ds(j * 16, 16)] = plsc.load_gather(yt_v, [row16])
                zb_v[b][pl.ds(j * 16, 16)] = plsc.load_gather(zt_v, [row16])
                return carry
            lax.fori_loop(0, ch // 16, xyz_body, 0)

        idx_cp(0, 0).start()
        idx_cp(0, 0).wait()
        gather_cp(0).start()
        xyz_comp(0)
        idx_cp(1, 1).start()
        idx_cp(1, 1).wait()
        gather_cp(1).start()
        xyz_comp(1)
        idx_cp(2, 2).start()
        gather_cp(0).wait()
        for cp in out_cps(0, 0):
            cp.start()
        idx_cp(2, 2).wait()
        gather_cp(2).start()
        xyz_comp(2)
        idx_cp(3, 0).start()
        gather_cp(1).wait()
        for cp in out_cps(1, 1):
            cp.start()

        def steady(j, carry):
            for b in range(3):
                c = 3 * j + b
                for cp in out_cps(c - 3, b):
                    cp.wait()
                idx_cp(c, b).wait()
                gather_cp(b).start()
                xyz_comp(b)
                bn1 = (b + 1) % 3
                if b < 2:
                    idx_cp(c + 1, bn1).start()
                else:
                    @pl.when(j < n_grp - 1)
                    def _():
                        idx_cp(c + 1, bn1).start()
                bp = (b + 2) % 3
                gather_cp(bp).wait()
                for cp in out_cps(c - 1, bp):
                    cp.start()
            return carry

        lax.fori_loop(1, n_grp, steady, 0)

        last = n_full - 1
        gather_cp(2).wait()
        for cp in out_cps(last, 2):
            cp.start()
        for b, c in ((0, last - 2), (1, last - 1), (2, last)):
            for cp in out_cps(c, b):
                cp.wait()

        if rem:
            base = base0 + n_full * ch
            sl = pl.ds(base, rem)
            pltpu.sync_copy(idx_hbm.at[sl], idx_v[0].at[pl.ds(0, rem)])
            pltpu.async_copy(
                feats_hbm.at[idx_v[0].at[pl.ds(0, rem)]],
                rows_v[0].at[pl.ds(0, rem)], sem_g[0]).wait()

            def rem_body(j, carry):
                row16 = idx_v[0][pl.ds(j * 16, 16)]
                xb_v[0][pl.ds(j * 16, 16)] = plsc.load_gather(xt_v, [row16])
                yb_v[0][pl.ds(j * 16, 16)] = plsc.load_gather(yt_v, [row16])
                zb_v[0][pl.ds(j * 16, 16)] = plsc.load_gather(zt_v, [row16])
                return carry
            lax.fori_loop(0, (rem + 15) // 16, rem_body, 0)
            pltpu.sync_copy(rows_v[0].at[pl.ds(0, rem)], gf_hbm.at[sl])
            pltpu.sync_copy(xb_v[0].at[pl.ds(0, rem)], gx_hbm.at[sl])
            pltpu.sync_copy(yb_v[0].at[pl.ds(0, rem)], gy_hbm.at[sl])
            pltpu.sync_copy(zb_v[0].at[pl.ds(0, rem)], gz_hbm.at[sl])

    return gather


def _tc_main_body(gf_ref, gx_ref, gy_ref, gz_ref, qx_ref, qy_ref, qz_ref,
                  kp_ref, w_ref, out_ref, sums_ref, gall_ref,
                  *, bq, k, c_in, c_out, p):
    i = pl.program_id(0)
    ng = (bq * k) // 256

    kxb = jnp.broadcast_to(kp_ref[:, 0:1], (16, 256))
    kyb = jnp.broadcast_to(kp_ref[:, 1:2], (16, 256))
    kzb = jnp.broadcast_to(kp_ref[:, 2:3], (16, 256))
    sub = lax.broadcasted_iota(jnp.int32, (128, 256), 0)
    lan = lax.broadcasted_iota(jnp.int32, (128, 256), 1)
    mask = ((sub % 8) == (lan // 32)).astype(jnp.float32)

    for g in range(ng):
        ex = gx_ref[0, g:g + 1, :] - qx_ref[0, g:g + 1, :]
        ey = gy_ref[0, g:g + 1, :] - qy_ref[0, g:g + 1, :]
        ez = gz_ref[0, g:g + 1, :] - qz_ref[0, g:g + 1, :]
        tx = jnp.broadcast_to(ex, (16, 256)) - kxb
        ty = jnp.broadcast_to(ey, (16, 256)) - kyb
        tz = jnp.broadcast_to(ez, (16, 256)) - kzb
        sq = tx * tx + ty * ty + tz * tz
        dist = sq * lax.rsqrt(sq + 1e-12)
        infl = jnp.maximum(1.0 - dist * (1.0 / SIGMA), 0.0)
        rep = jnp.broadcast_to(
            infl.reshape(16, 1, 256), (16, 8, 256)).reshape(128, 256)
        bdt = rep * mask
        g2 = jnp.dot(bdt, gf_ref[g * 256:(g + 1) * 256, :],
                     preferred_element_type=jnp.float32)
        for pp in range(p):
            gall_ref[pp * bq + g * 8:pp * bq + g * 8 + 8, :] = (
                g2[pp * 8:(pp + 1) * 8, :])

    acc = jnp.zeros((bq, c_out), jnp.float32)
    for pp in range(p):
        acc = acc + jnp.dot(gall_ref[pp * bq:(pp + 1) * bq, :], w_ref[pp],
                            preferred_element_type=jnp.float32)
    out_ref[...] = acc

    @pl.when(i == 0)
    def _():
        sums_ref[...] = jnp.zeros_like(sums_ref)

    sums_ref[0:1, :] += jnp.sum(acc, axis=0, keepdims=True)
    sums_ref[1:2, :] += jnp.sum(acc * acc, axis=0, keepdims=True)


def _tc_norm_body(x_ref, *refs, n, c_out):
    sums_refs = refs[:-3]
    gamma_ref, beta_ref, y_ref = refs[-3:]
    cpg = c_out // GROUPS
    r = lax.broadcasted_iota(jnp.int32, (c_out, c_out), 0) // cpg
    c = lax.broadcasted_iota(jnp.int32, (c_out, c_out), 1) // cpg
    gmat = (r == c).astype(jnp.float32)
    csum = sum(s[0:1, :] for s in sums_refs)
    csq = sum(s[1:2, :] for s in sums_refs)
    denom = 1.0 / (n * cpg)
    mean = jnp.dot(csum, gmat, preferred_element_type=jnp.float32) * denom
    e2 = jnp.dot(csq, gmat, preferred_element_type=jnp.float32) * denom
    var = e2 - mean * mean
    a = gamma_ref[...] * lax.rsqrt(var + EPS)
    b = beta_ref[...] - mean * a
    y = x_ref[...] * a + b
    y_ref[...] = jnp.where(y >= 0, y, NEG_SLOPE * y)


def kernel(q_points, s_points, s_feats, neighbor_indices, kernel_points,
           weights, gamma, beta):
    n, c_in = s_feats.shape
    nq, k = neighbor_indices.shape
    p, _, c_out = weights.shape
    n_edges = nq * k

    idx_flat = neighbor_indices.reshape(n_edges)
    kp_t = jnp.full((16, 8), 1e4, jnp.float32).at[0:p, 0:3].set(kernel_points)
    bq = 200
    n_slices = 1
    nq_s = nq // n_slices
    ne_s = n_edges // n_slices
    n_blocks = nq_s // bq
    ng = (bq * k) // 256
    e3 = (n_blocks, ng, 256)

    gather = _make_sc_gather(n, ne_s, c_in, jnp.float32)
    sx, sy, sz = s_points[:, 0], s_points[:, 1], s_points[:, 2]

    outs, sums_l = [], []
    for s in range(n_slices):
        idx_s = idx_flat[s * ne_s:(s + 1) * ne_s]
        if s >= 2:
            idx_s, _ = lax.optimization_barrier((idx_s, outs[s - 2]))
        gf, gxf, gyf, gzf = gather(s_feats, sx, sy, sz, idx_s)
        gx = gxf.reshape(e3)
        gy = gyf.reshape(e3)
        gz = gzf.reshape(e3)
        qsl = q_points[s * nq_s:(s + 1) * nq_s]
        qx = jnp.repeat(qsl[:, 0], k).reshape(e3)
        qy = jnp.repeat(qsl[:, 1], k).reshape(e3)
        qz = jnp.repeat(qsl[:, 2], k).reshape(e3)

        out_s, sums_s = pl.pallas_call(
            functools.partial(_tc_main_body, bq=bq, k=k, c_in=c_in,
                              c_out=c_out, p=p),
            grid=(n_blocks,),
            in_specs=[
                pl.BlockSpec((bq * k, c_in), lambda i: (i, 0)),
                pl.BlockSpec((1, ng, 256), lambda i: (i, 0, 0)),
                pl.BlockSpec((1, ng, 256), lambda i: (i, 0, 0)),
                pl.BlockSpec((1, ng, 256), lambda i: (i, 0, 0)),
                pl.BlockSpec((1, ng, 256), lambda i: (i, 0, 0)),
                pl.BlockSpec((1, ng, 256), lambda i: (i, 0, 0)),
                pl.BlockSpec((1, ng, 256), lambda i: (i, 0, 0)),
                pl.BlockSpec((16, 8), lambda i: (0, 0)),
                pl.BlockSpec((p, c_in, c_out), lambda i: (0, 0, 0)),
            ],
            out_specs=[
                pl.BlockSpec((bq, c_out), lambda i: (i, 0)),
                pl.BlockSpec((8, c_out), lambda i: (0, 0)),
            ],
            out_shape=[
                jax.ShapeDtypeStruct((nq_s, c_out), jnp.float32),
                jax.ShapeDtypeStruct((8, c_out), jnp.float32),
            ],
            scratch_shapes=[pltpu.VMEM((p * bq, c_in), jnp.float32)],
        )(gf, gx, gy, gz, qx, qy, qz, kp_t, weights)
        outs.append(out_s)
        sums_l.append(sums_s)

    out = outs[0] if n_slices == 1 else jnp.concatenate(outs, axis=0)

    bn = 1000
    y = pl.pallas_call(
        functools.partial(_tc_norm_body, n=nq, c_out=c_out),
        grid=(nq // bn,),
        in_specs=[
            pl.BlockSpec((bn, c_out), lambda i: (i, 0)),
        ] + [
            pl.BlockSpec((8, c_out), lambda i: (0, 0))
            for _ in range(n_slices)
        ] + [
            pl.BlockSpec((1, c_out), lambda i: (0, 0)),
            pl.BlockSpec((1, c_out), lambda i: (0, 0)),
        ],
        out_specs=pl.BlockSpec((bn, c_out), lambda i: (i, 0)),
        out_shape=jax.ShapeDtypeStruct((nq, c_out), jnp.float32),
    )(out, *sums_l, gamma.reshape(1, c_out), beta.reshape(1, c_out))

    return y


# speedup vs baseline: 1.2971x; 1.0033x over previous
import functools

import jax
import jax.numpy as jnp
from jax import lax
from jax.experimental import pallas as pl
from jax.experimental.pallas import tpu as pltpu
from jax.experimental.pallas import tpu_sc as plsc

SIGMA = 2.0
EPS = 1e-5
NEG_SLOPE = 0.01
GROUPS = 32


def _make_sc_gather(n_rows, n_edges, c_feat, feat_dtype):
    info = plsc.get_sparse_core_info()
    nc, ns = info.num_cores, info.num_subcores
    nw = nc * ns
    assert n_edges % nw == 0
    per_w = n_edges // nw
    ch = 128
    n_full = per_w // ch
    rem = per_w - n_full * ch
    assert rem % 8 == 0 and n_full % 3 == 0 and n_full >= 6
    n_grp = n_full // 3
    mesh = plsc.VectorSubcoreMesh(core_axis_name="c", subcore_axis_name="s")

    nb = 3
    scratch = (
        [pltpu.VMEM((ch,), jnp.int32)] * nb
        + [pltpu.VMEM((ch, c_feat), feat_dtype)] * nb
        + [pltpu.VMEM((ch,), jnp.float32)] * (3 * nb)
        + [
            pltpu.VMEM((n_rows,), jnp.float32),
            pltpu.VMEM((n_rows,), jnp.float32),
            pltpu.VMEM((n_rows,), jnp.float32),
        ]
        + [pltpu.SemaphoreType.DMA] * (3 * nb)
    )

    @functools.partial(
        pl.kernel,
        mesh=mesh,
        compiler_params=pltpu.CompilerParams(needs_layout_passes=False),
        out_type=[
            jax.ShapeDtypeStruct((n_edges, c_feat), feat_dtype),
            jax.ShapeDtypeStruct((n_edges,), jnp.float32),
            jax.ShapeDtypeStruct((n_edges,), jnp.float32),
            jax.ShapeDtypeStruct((n_edges,), jnp.float32),
        ],
        scratch_types=scratch,
    )
    def gather(feats_hbm, xs_hbm, ys_hbm, zs_hbm, idx_hbm,
               gf_hbm, gx_hbm, gy_hbm, gz_hbm, *bufs):
        idx_v = bufs[0:nb]
        rows_v = bufs[nb:2 * nb]
        xb_v = bufs[2 * nb:3 * nb]
        yb_v = bufs[3 * nb:4 * nb]
        zb_v = bufs[4 * nb:5 * nb]
        xt_v, yt_v, zt_v = bufs[5 * nb:5 * nb + 3]
        sem_i = bufs[5 * nb + 3:5 * nb + 3 + nb]
        sem_g = bufs[5 * nb + 3 + nb:5 * nb + 3 + 2 * nb]
        sem_o = bufs[5 * nb + 3 + 2 * nb:5 * nb + 3 + 3 * nb]

        wid = lax.axis_index("s") * nc + lax.axis_index("c")
        base0 = wid * per_w
        pltpu.sync_copy(xs_hbm, xt_v)
        pltpu.sync_copy(ys_hbm, yt_v)
        pltpu.sync_copy(zs_hbm, zt_v)

        def idx_cp(c, b):
            return pltpu.make_async_copy(
                idx_hbm.at[pl.ds(base0 + c * ch, ch)], idx_v[b], sem_i[b])

        def gather_cp(b):
            return pltpu.make_async_copy(
                feats_hbm.at[idx_v[b]], rows_v[b], sem_g[b])

        def out_cps(c, b):
            sl = pl.ds(base0 + c * ch, ch)
            return (
                pltpu.make_async_copy(rows_v[b], gf_hbm.at[sl], sem_o[b]),
                pltpu.make_async_copy(xb_v[b], gx_hbm.at[sl], sem_o[b]),
                pltpu.make_async_copy(yb_v[b], gy_hbm.at[sl], sem_o[b]),
                pltpu.make_async_copy(zb_v[b], gz_hbm.at[sl], sem_o[b]),
            )

        def xyz_comp(b):
            def xyz_body(j, carry):
                row16 = idx_v[b][pl.ds(j * 16, 16)]
                xb_v[b][pl.ds(j * 16, 16)] = plsc.load_gather(xt_v, [row16])
                yb_v[b][pl.ds(j * 16, 16)] = plsc.load_gather(yt_v, [row16])
                zb_v[b][pl.ds(j * 16, 16)] = plsc.load_gather(zt_v, [row16])
                return carry
            lax.fori_loop(0, ch // 16, xyz_body, 0)

        idx_cp(0, 0).start()
        idx_cp(0, 0).wait()
        gather_cp(0).start()
        xyz_comp(0)
        idx_cp(1, 1).start()
        idx_cp(1, 1).wait()
        gather_cp(1).start()
        xyz_comp(1)
        idx_cp(2, 2).start()
        gather_cp(0).wait()
        for cp in out_cps(0, 0):
            cp.start()
        idx_cp(2, 2).wait()
        gather_cp(2).start()
        xyz_comp(2)
        idx_cp(3, 0).start()
        gather_cp(1).wait()
        for cp in out_cps(1, 1):
            cp.start()

        def steady(j, carry):
            for b in range(3):
                c = 3 * j + b
                for cp in out_cps(c - 3, b):
                    cp.wait()
                idx_cp(c, b).wait()
                gather_cp(b).start()
                xyz_comp(b)
                bn1 = (b + 1) % 3
                if b < 2:
                    idx_cp(c + 1, bn1).start()
                else:
                    @pl.when(j < n_grp - 1)
                    def _():
                        idx_cp(c + 1, bn1).start()
                bp = (b + 2) % 3
                gather_cp(bp).wait()
                for cp in out_cps(c - 1, bp):
                    cp.start()
            return carry

        lax.fori_loop(1, n_grp, steady, 0)

        last = n_full - 1
        gather_cp(2).wait()
        for cp in out_cps(last, 2):
            cp.start()
        for b, c in ((0, last - 2), (1, last - 1), (2, last)):
            for cp in out_cps(c, b):
                cp.wait()

        if rem:
            base = base0 + n_full * ch
            sl = pl.ds(base, rem)
            pltpu.sync_copy(idx_hbm.at[sl], idx_v[0].at[pl.ds(0, rem)])
            pltpu.async_copy(
                feats_hbm.at[idx_v[0].at[pl.ds(0, rem)]],
                rows_v[0].at[pl.ds(0, rem)], sem_g[0]).wait()

            def rem_body(j, carry):
                row16 = idx_v[0][pl.ds(j * 16, 16)]
                xb_v[0][pl.ds(j * 16, 16)] = plsc.load_gather(xt_v, [row16])
                yb_v[0][pl.ds(j * 16, 16)] = plsc.load_gather(yt_v, [row16])
                zb_v[0][pl.ds(j * 16, 16)] = plsc.load_gather(zt_v, [row16])
                return carry
            lax.fori_loop(0, (rem + 15) // 16, rem_body, 0)
            pltpu.sync_copy(rows_v[0].at[pl.ds(0, rem)], gf_hbm.at[sl])
            pltpu.sync_copy(xb_v[0].at[pl.ds(0, rem)], gx_hbm.at[sl])
            pltpu.sync_copy(yb_v[0].at[pl.ds(0, rem)], gy_hbm.at[sl])
            pltpu.sync_copy(zb_v[0].at[pl.ds(0, rem)], gz_hbm.at[sl])

    return gather


def _tc_main_body(gf_ref, gx_ref, gy_ref, gz_ref, qx_ref, qy_ref, qz_ref,
                  kp_ref, w_ref, out_ref, sums_ref, gall_ref,
                  *, bq, k, c_in, c_out, p):
    i = pl.program_id(0)
    ng = (bq * k) // 256

    kxb = jnp.broadcast_to(kp_ref[:, 0:1], (16, 256))
    kyb = jnp.broadcast_to(kp_ref[:, 1:2], (16, 256))
    kzb = jnp.broadcast_to(kp_ref[:, 2:3], (16, 256))
    sub = lax.broadcasted_iota(jnp.int32, (128, 256), 0)
    lan = lax.broadcasted_iota(jnp.int32, (128, 256), 1)
    mask = ((sub % 8) == (lan // 32)).astype(jnp.float32)

    for g in range(ng):
        ex = gx_ref[0, g:g + 1, :] - qx_ref[0, g:g + 1, :]
        ey = gy_ref[0, g:g + 1, :] - qy_ref[0, g:g + 1, :]
        ez = gz_ref[0, g:g + 1, :] - qz_ref[0, g:g + 1, :]
        tx = jnp.broadcast_to(ex, (16, 256)) - kxb
        ty = jnp.broadcast_to(ey, (16, 256)) - kyb
        tz = jnp.broadcast_to(ez, (16, 256)) - kzb
        sq = tx * tx + ty * ty + tz * tz
        dist = sq * lax.rsqrt(sq + 1e-12)
        infl = jnp.maximum(1.0 - dist * (1.0 / SIGMA), 0.0)
        rep = jnp.broadcast_to(
            infl.reshape(16, 1, 256), (16, 8, 256)).reshape(128, 256)
        bdt = rep * mask
        g2 = jnp.dot(bdt, gf_ref[g * 256:(g + 1) * 256, :],
                     preferred_element_type=jnp.float32)
        for pp in range(p):
            gall_ref[pp * bq + g * 8:pp * bq + g * 8 + 8, :] = (
                g2[pp * 8:(pp + 1) * 8, :])

    acc = jnp.zeros((bq, c_out), jnp.float32)
    for pp in range(p):
        acc = acc + jnp.dot(gall_ref[pp * bq:(pp + 1) * bq, :], w_ref[pp],
                            preferred_element_type=jnp.float32)
    out_ref[...] = acc

    @pl.when(i == 0)
    def _():
        sums_ref[...] = jnp.zeros_like(sums_ref)

    sums_ref[0:1, :] += jnp.sum(acc, axis=0, keepdims=True)
    sums_ref[1:2, :] += jnp.sum(acc * acc, axis=0, keepdims=True)


def _tc_norm_body(x_ref, *refs, n, c_out):
    sums_refs = refs[:-3]
    gamma_ref, beta_ref, y_ref = refs[-3:]
    cpg = c_out // GROUPS
    r = lax.broadcasted_iota(jnp.int32, (c_out, c_out), 0) // cpg
    c = lax.broadcasted_iota(jnp.int32, (c_out, c_out), 1) // cpg
    gmat = (r == c).astype(jnp.float32)
    csum = sum(s[0:1, :] for s in sums_refs)
    csq = sum(s[1:2, :] for s in sums_refs)
    denom = 1.0 / (n * cpg)
    mean = jnp.dot(csum, gmat, preferred_element_type=jnp.float32) * denom
    e2 = jnp.dot(csq, gmat, preferred_element_type=jnp.float32) * denom
    var = e2 - mean * mean
    a = gamma_ref[...] * lax.rsqrt(var + EPS)
    b = beta_ref[...] - mean * a
    y = x_ref[...] * a + b
    y_ref[...] = jnp.where(y >= 0, y, NEG_SLOPE * y)


def kernel(q_points, s_points, s_feats, neighbor_indices, kernel_points,
           weights, gamma, beta):
    n, c_in = s_feats.shape
    nq, k = neighbor_indices.shape
    p, _, c_out = weights.shape
    n_edges = nq * k

    idx_flat = neighbor_indices.reshape(n_edges)
    kp_t = jnp.full((16, 8), 1e4, jnp.float32).at[0:p, 0:3].set(kernel_points)
    bq = 200
    n_slices = 1
    nq_s = nq // n_slices
    ne_s = n_edges // n_slices
    n_blocks = nq_s // bq
    ng = (bq * k) // 256
    e3 = (n_blocks, ng, 256)

    gather = _make_sc_gather(n, ne_s, c_in, jnp.float32)
    sx, sy, sz = s_points[:, 0], s_points[:, 1], s_points[:, 2]

    outs, sums_l = [], []
    for s in range(n_slices):
        idx_s = idx_flat[s * ne_s:(s + 1) * ne_s]
        if s >= 2:
            idx_s, _ = lax.optimization_barrier((idx_s, outs[s - 2]))
        gf, gxf, gyf, gzf = gather(s_feats, sx, sy, sz, idx_s)
        gx = gxf.reshape(e3)
        gy = gyf.reshape(e3)
        gz = gzf.reshape(e3)
        qsl = q_points[s * nq_s:(s + 1) * nq_s]
        qx = jnp.repeat(qsl[:, 0], k).reshape(e3)
        qy = jnp.repeat(qsl[:, 1], k).reshape(e3)
        qz = jnp.repeat(qsl[:, 2], k).reshape(e3)

        out_s, sums_s = pl.pallas_call(
            functools.partial(_tc_main_body, bq=bq, k=k, c_in=c_in,
                              c_out=c_out, p=p),
            grid=(n_blocks,),
            in_specs=[
                pl.BlockSpec((bq * k, c_in), lambda i: (i, 0)),
                pl.BlockSpec((1, ng, 256), lambda i: (i, 0, 0)),
                pl.BlockSpec((1, ng, 256), lambda i: (i, 0, 0)),
                pl.BlockSpec((1, ng, 256), lambda i: (i, 0, 0)),
                pl.BlockSpec((1, ng, 256), lambda i: (i, 0, 0)),
                pl.BlockSpec((1, ng, 256), lambda i: (i, 0, 0)),
                pl.BlockSpec((1, ng, 256), lambda i: (i, 0, 0)),
                pl.BlockSpec((16, 8), lambda i: (0, 0)),
                pl.BlockSpec((p, c_in, c_out), lambda i: (0, 0, 0)),
            ],
            out_specs=[
                pl.BlockSpec((bq, c_out), lambda i: (i, 0)),
                pl.BlockSpec((8, c_out), lambda i: (0, 0)),
            ],
            out_shape=[
                jax.ShapeDtypeStruct((nq_s, c_out), jnp.float32),
                jax.ShapeDtypeStruct((8, c_out), jnp.float32),
            ],
            scratch_shapes=[pltpu.VMEM((p * bq, c_in), jnp.float32)],
        )(gf, gx, gy, gz, qx, qy, qz, kp_t, weights)
        outs.append(out_s)
        sums_l.append(sums_s)

    out = outs[0] if n_slices == 1 else jnp.concatenate(outs, axis=0)

    bn = 2000
    y = pl.pallas_call(
        functools.partial(_tc_norm_body, n=nq, c_out=c_out),
        grid=(nq // bn,),
        in_specs=[
            pl.BlockSpec((bn, c_out), lambda i: (i, 0)),
        ] + [
            pl.BlockSpec((8, c_out), lambda i: (0, 0))
            for _ in range(n_slices)
        ] + [
            pl.BlockSpec((1, c_out), lambda i: (0, 0)),
            pl.BlockSpec((1, c_out), lambda i: (0, 0)),
        ],
        out_specs=pl.BlockSpec((bn, c_out), lambda i: (i, 0)),
        out_shape=jax.ShapeDtypeStruct((nq, c_out), jnp.float32),
    )(out, *sums_l, gamma.reshape(1, c_out), beta.reshape(1, c_out))

    return y
